# Initial kernel scaffold; baseline (speedup 1.0000x reference)
import jax
import jax.numpy as jnp
from jax.experimental import pallas as pl


def kernel(x, edge_index, edge_feat, W1, b1, W2, b2, W3, b3):
    raise NotImplementedError("write your pallas kernel here")


# speedup vs baseline: 2.2174x; 2.2174x over previous
import functools

import jax
import jax.numpy as jnp
from jax import lax
from jax.experimental import pallas as pl
from jax.experimental.pallas import tpu as pltpu
from jax.experimental.pallas import tpu_sc as plsc

_NC = 2
_NS = 16
_NW = _NC * _NS
_CHUNK = 80

_PREC = jax.lax.Precision.HIGHEST


def _npad_for(n):
    blk = _NS * _CHUNK
    return ((n + blk - 1) // blk) * blk


def _proj_body(x_ref, w_ref, xs_ref, xt_ref):
    xb = x_ref[...]
    w = w_ref[...]
    d = xb.shape[1]
    xs_ref[...] = jnp.dot(xb, w[:d, :], precision=_PREC,
                          preferred_element_type=jnp.float32)
    xt_ref[...] = jnp.dot(xb, w[d:, :], precision=_PREC,
                          preferred_element_type=jnp.float32)


def _node_proj(x, w_ab):
    n, d = x.shape
    h = w_ab.shape[1]
    bn = 2000
    grid = (n // bn,)
    return pl.pallas_call(
        _proj_body,
        grid=grid,
        in_specs=[
            pl.BlockSpec((bn, d), lambda i: (i, 0)),
            pl.BlockSpec((2 * d, h), lambda i: (0, 0)),
        ],
        out_specs=[
            pl.BlockSpec((bn, h), lambda i: (i, 0)),
            pl.BlockSpec((bn, h), lambda i: (i, 0)),
        ],
        out_shape=[
            jax.ShapeDtypeStruct((n, h), jnp.float32),
            jax.ShapeDtypeStruct((n, h), jnp.float32),
        ],
    )(x, w_ab)


def _gather_body(e_tot, h, xs_hbm, xt_hbm, src_hbm, dst_hbm, out_hbm,
                 idx_s, idx_t, rows_a, rows_b, sem_a, sem_b):
    c = lax.axis_index("c")
    s = lax.axis_index("s")
    wid = s * _NC + c
    ew = e_tot // _NW
    nvec = h // 16

    def chunk(i, carry):
        base = wid * ew + i * _CHUNK
        pltpu.sync_copy(src_hbm.at[pl.ds(base, _CHUNK)], idx_s)
        pltpu.sync_copy(dst_hbm.at[pl.ds(base, _CHUNK)], idx_t)
        ca = pltpu.async_copy(xs_hbm.at[idx_s], rows_a, sem_a)
        cb = pltpu.async_copy(xt_hbm.at[idx_t], rows_b, sem_b)
        ca.wait()
        cb.wait()

        def row(r, rc):
            for j in range(nvec):
                sl = pl.ds(j * 16, 16)
                rows_a[r, sl] = rows_a[r, sl] + rows_b[r, sl]
            return rc

        lax.fori_loop(0, _CHUNK, row, 0)
        pltpu.sync_copy(rows_a, out_hbm.at[pl.ds(base, _CHUNK)])
        return carry

    lax.fori_loop(0, ew // _CHUNK, chunk, 0)


def _gather_add(xs, xt, src, dst):
    n, h = xs.shape
    e_tot = src.shape[0]
    mesh = plsc.VectorSubcoreMesh(core_axis_name="c", subcore_axis_name="s")
    kern = pl.kernel(
        functools.partial(_gather_body, e_tot, h),
        out_type=jax.ShapeDtypeStruct((e_tot, h), jnp.float32),
        mesh=mesh,
        scratch_types=[
            pltpu.VMEM((_CHUNK,), jnp.int32),
            pltpu.VMEM((_CHUNK,), jnp.int32),
            pltpu.VMEM((_CHUNK, h), jnp.float32),
            pltpu.VMEM((_CHUNK, h), jnp.float32),
            pltpu.SemaphoreType.DMA,
            pltpu.SemaphoreType.DMA,
        ],
    )
    return kern(xs, xt, src, dst)


def _mlp_body(pre_ref, ef_ref, w1c_ref, b1_ref, w2_ref, b2_ref, w3_ref, b3_ref,
              msg_ref):
    pre = pre_ref[...]
    ef = ef_ref[...]
    h1 = pre + jnp.dot(ef, w1c_ref[...], precision=_PREC,
                       preferred_element_type=jnp.float32) + b1_ref[...]
    h1 = jnp.maximum(h1, 0.0)
    h2 = jnp.dot(h1, w2_ref[...], precision=_PREC,
                 preferred_element_type=jnp.float32) + b2_ref[...]
    h2 = jnp.maximum(h2, 0.0)
    msg_ref[...] = jnp.dot(h2, w3_ref[...], precision=_PREC,
                           preferred_element_type=jnp.float32) + b3_ref[...]


def _edge_mlp(pre, ef, w1c, b1, w2, b2, w3, b3):
    e_tot, h = pre.shape
    de = ef.shape[1]
    d_out = w3.shape[1]
    be = 4000
    grid = (e_tot // be,)
    b1r = b1.reshape(1, h)
    b2r = b2.reshape(1, h)
    b3r = b3.reshape(1, d_out)
    return pl.pallas_call(
        _mlp_body,
        grid=grid,
        in_specs=[
            pl.BlockSpec((be, h), lambda i: (i, 0)),
            pl.BlockSpec((be, de), lambda i: (i, 0)),
            pl.BlockSpec((de, h), lambda i: (0, 0)),
            pl.BlockSpec((1, h), lambda i: (0, 0)),
            pl.BlockSpec((h, h), lambda i: (0, 0)),
            pl.BlockSpec((1, h), lambda i: (0, 0)),
            pl.BlockSpec((h, d_out), lambda i: (0, 0)),
            pl.BlockSpec((1, d_out), lambda i: (0, 0)),
        ],
        out_specs=pl.BlockSpec((be, d_out), lambda i: (i, 0)),
        out_shape=jax.ShapeDtypeStruct((e_tot, d_out), jnp.float32),
    )(pre, ef, w1c, b1r, w2, b2r, w3, b3r)


def _sums_body(e_tot, npad, d, msg_hbm, tgt_hbm, sums_hbm,
               msg_v, idx_v, sum_tbl):
    c = lax.axis_index("c")
    s = lax.axis_index("s")
    wid = s * _NC + c
    ew = e_tot // _NW
    rpt = npad // _NS
    kper = rpt // _CHUNK

    def zrow(r, carry):
        for j in range(d // 16):
            msg_v[r, pl.ds(j * 16, 16)] = jnp.zeros((16,), jnp.float32)
        return carry

    lax.fori_loop(0, _CHUNK, zrow, 0)
    for k in range(kper):
        sl = pl.ds(s * rpt + k * _CHUNK, _CHUNK)
        pltpu.sync_copy(msg_v, sum_tbl.at[sl])
    plsc.subcore_barrier()

    def chunk(i, carry):
        base = wid * ew + i * _CHUNK
        pltpu.sync_copy(tgt_hbm.at[pl.ds(base, _CHUNK)], idx_v)
        pltpu.sync_copy(msg_hbm.at[pl.ds(base, _CHUNK)], msg_v)
        pltpu.sync_copy(msg_v, sum_tbl.at[idx_v], add=True)
        return carry

    lax.fori_loop(0, ew // _CHUNK, chunk, 0)
    plsc.subcore_barrier()

    for k in range(kper):
        sl = pl.ds(s * rpt + k * _CHUNK, _CHUNK)
        osl = pl.ds(c * npad + s * rpt + k * _CHUNK, _CHUNK)
        pltpu.sync_copy(sum_tbl.at[sl], msg_v)
        pltpu.sync_copy(msg_v, sums_hbm.at[osl])


def _scatter_sums(msg, tgt, npad):
    e_tot, d = msg.shape
    mesh = plsc.VectorSubcoreMesh(core_axis_name="c", subcore_axis_name="s")
    kern = pl.kernel(
        functools.partial(_sums_body, e_tot, npad, d),
        out_type=jax.ShapeDtypeStruct((_NC * npad, d), jnp.float32),
        mesh=mesh,
        scratch_types=[
            pltpu.VMEM((_CHUNK, d), jnp.float32),
            pltpu.VMEM((_CHUNK,), jnp.int32),
            pltpu.VMEM_SHARED((npad, d), jnp.float32),
        ],
    )
    return kern(msg, tgt)


def _cnts_body(e_tot, npad, d, tgt_hbm, cnts_hbm, ones_v, idx_v, cnt_tbl):
    c = lax.axis_index("c")
    s = lax.axis_index("s")
    wid = s * _NC + c
    ew = e_tot // _NW
    rpt = npad // _NS
    kper = rpt // _CHUNK
    one16 = jnp.full((16,), 1.0, jnp.float32)
    zero16 = jnp.zeros((16,), jnp.float32)

    def zrow(r, carry):
        for j in range(d // 16):
            ones_v[r, pl.ds(j * 16, 16)] = zero16
        return carry

    lax.fori_loop(0, _CHUNK, zrow, 0)
    for k in range(kper):
        sl = pl.ds(s * rpt + k * _CHUNK, _CHUNK)
        pltpu.sync_copy(ones_v, cnt_tbl.at[sl])

    def onerow(r, carry):
        for j in range(d // 16):
            ones_v[r, pl.ds(j * 16, 16)] = one16
        return carry

    lax.fori_loop(0, _CHUNK, onerow, 0)
    plsc.subcore_barrier()

    def chunk(i, carry):
        base = wid * ew + i * _CHUNK
        pltpu.sync_copy(tgt_hbm.at[pl.ds(base, _CHUNK)], idx_v)
        pltpu.sync_copy(ones_v, cnt_tbl.at[idx_v], add=True)
        return carry

    lax.fori_loop(0, ew // _CHUNK, chunk, 0)
    plsc.subcore_barrier()

    for k in range(kper):
        sl = pl.ds(s * rpt + k * _CHUNK, _CHUNK)
        osl = pl.ds(c * npad + s * rpt + k * _CHUNK, _CHUNK)
        pltpu.sync_copy(cnt_tbl.at[sl], ones_v)
        pltpu.sync_copy(ones_v, cnts_hbm.at[osl])


def _scatter_counts(tgt, npad, d):
    e_tot = tgt.shape[0]
    mesh = plsc.VectorSubcoreMesh(core_axis_name="c", subcore_axis_name="s")
    kern = pl.kernel(
        functools.partial(_cnts_body, e_tot, npad, d),
        out_type=jax.ShapeDtypeStruct((_NC * npad, d), jnp.float32),
        mesh=mesh,
        scratch_types=[
            pltpu.VMEM((_CHUNK, d), jnp.float32),
            pltpu.VMEM((_CHUNK,), jnp.int32),
            pltpu.VMEM_SHARED((npad, d), jnp.float32),
        ],
    )
    return kern(tgt)


def _final_body(x_ref, s0_ref, s1_ref, c0_ref, c1_ref, o_ref):
    ssum = s0_ref[...] + s1_ref[...]
    cnt = c0_ref[...] + c1_ref[...]
    o_ref[...] = x_ref[...] + ssum / jnp.maximum(cnt, 1.0)


def _finalize(x, sums, cnts, npad):
    n, d = x.shape
    bn = 2000
    grid = ((n + bn - 1) // bn,)
    s0, s1 = sums[:npad], sums[npad:]
    c0, c1 = cnts[:npad], cnts[npad:]
    return pl.pallas_call(
        _final_body,
        grid=grid,
        in_specs=[
            pl.BlockSpec((bn, d), lambda i: (i, 0)),
            pl.BlockSpec((bn, d), lambda i: (i, 0)),
            pl.BlockSpec((bn, d), lambda i: (i, 0)),
            pl.BlockSpec((bn, d), lambda i: (i, 0)),
            pl.BlockSpec((bn, d), lambda i: (i, 0)),
        ],
        out_specs=pl.BlockSpec((bn, d), lambda i: (i, 0)),
        out_shape=jax.ShapeDtypeStruct((n, d), jnp.float32),
    )(x, s0, s1, c0, c1)


def kernel(x, edge_index, edge_feat, W1, b1, W2, b2, W3, b3):
    n, d = x.shape
    src = edge_index[0]
    dst = edge_index[1]
    w_ab = W1[: 2 * d, :]
    w1c = W1[2 * d:, :]
    npad = _npad_for(n)

    xs, xt = _node_proj(x, w_ab)
    pre = _gather_add(xs, xt, src, dst)
    cnts = _scatter_counts(dst, npad, d)
    msg = _edge_mlp(pre, edge_feat, w1c, b1, W2, b2, W3, b3)
    sums = _scatter_sums(msg, dst, npad)
    return _finalize(x, sums, cnts, npad)


# speedup vs baseline: 3.2791x; 1.4788x over previous
import functools

import jax
import jax.numpy as jnp
from jax import lax
from jax.experimental import pallas as pl
from jax.experimental.pallas import tpu as pltpu
from jax.experimental.pallas import tpu_sc as plsc

_NC = 2
_NS = 16
_NW = _NC * _NS
_CHUNK = 80

_PREC = None


def _npad_for(n):
    blk = _NS * _CHUNK
    return ((n + blk - 1) // blk) * blk


def _proj_body(x_ref, w_ref, xs_ref, xt_ref):
    xb = x_ref[...]
    w = w_ref[...]
    d = xb.shape[1]
    xs_ref[...] = jnp.dot(xb, w[:d, :], precision=_PREC,
                          preferred_element_type=jnp.float32)
    xt_ref[...] = jnp.dot(xb, w[d:, :], precision=_PREC,
                          preferred_element_type=jnp.float32)


def _node_proj(x, w_ab):
    n, d = x.shape
    h = w_ab.shape[1]
    bn = 2000
    grid = (n // bn,)
    return pl.pallas_call(
        _proj_body,
        grid=grid,
        in_specs=[
            pl.BlockSpec((bn, d), lambda i: (i, 0)),
            pl.BlockSpec((2 * d, h), lambda i: (0, 0)),
        ],
        out_specs=[
            pl.BlockSpec((bn, h), lambda i: (i, 0)),
            pl.BlockSpec((bn, h), lambda i: (i, 0)),
        ],
        out_shape=[
            jax.ShapeDtypeStruct((n, h), jnp.float32),
            jax.ShapeDtypeStruct((n, h), jnp.float32),
        ],
    )(x, w_ab)


def _gather_body(e_tot, h, xs_hbm, xt_hbm, src_hbm, dst_hbm, out_hbm,
                 idx_s, idx_t, rows_a, rows_b, sem_a, sem_b):
    c = lax.axis_index("c")
    s = lax.axis_index("s")
    wid = s * _NC + c
    ew = e_tot // _NW
    nvec = h // 16

    def chunk(i, carry):
        base = wid * ew + i * _CHUNK
        pltpu.sync_copy(src_hbm.at[pl.ds(base, _CHUNK)], idx_s)
        pltpu.sync_copy(dst_hbm.at[pl.ds(base, _CHUNK)], idx_t)
        ca = pltpu.async_copy(xs_hbm.at[idx_s], rows_a, sem_a)
        cb = pltpu.async_copy(xt_hbm.at[idx_t], rows_b, sem_b)
        ca.wait()
        cb.wait()

        def row(r, rc):
            for j in range(nvec):
                sl = pl.ds(j * 16, 16)
                rows_a[r, sl] = rows_a[r, sl] + rows_b[r, sl]
            return rc

        lax.fori_loop(0, _CHUNK, row, 0)
        pltpu.sync_copy(rows_a, out_hbm.at[pl.ds(base, _CHUNK)])
        return carry

    lax.fori_loop(0, ew // _CHUNK, chunk, 0)


def _gather_add(xs, xt, src, dst):
    n, h = xs.shape
    e_tot = src.shape[0]
    mesh = plsc.VectorSubcoreMesh(core_axis_name="c", subcore_axis_name="s")
    kern = pl.kernel(
        functools.partial(_gather_body, e_tot, h),
        out_type=jax.ShapeDtypeStruct((e_tot, h), jnp.float32),
        mesh=mesh,
        scratch_types=[
            pltpu.VMEM((_CHUNK,), jnp.int32),
            pltpu.VMEM((_CHUNK,), jnp.int32),
            pltpu.VMEM((_CHUNK, h), jnp.float32),
            pltpu.VMEM((_CHUNK, h), jnp.float32),
            pltpu.SemaphoreType.DMA,
            pltpu.SemaphoreType.DMA,
        ],
    )
    return kern(xs, xt, src, dst)


def _mlp_body(pre_ref, ef_ref, w1c_ref, b1_ref, w2_ref, b2_ref, w3_ref, b3_ref,
              msg_ref):
    pre = pre_ref[...]
    ef = ef_ref[...]
    h1 = pre + jnp.dot(ef, w1c_ref[...], precision=_PREC,
                       preferred_element_type=jnp.float32) + b1_ref[...]
    h1 = jnp.maximum(h1, 0.0)
    h2 = jnp.dot(h1, w2_ref[...], precision=_PREC,
                 preferred_element_type=jnp.float32) + b2_ref[...]
    h2 = jnp.maximum(h2, 0.0)
    msg_ref[...] = jnp.dot(h2, w3_ref[...], precision=_PREC,
                           preferred_element_type=jnp.float32) + b3_ref[...]


def _edge_mlp(pre, ef, w1c, b1, w2, b2, w3, b3):
    e_tot, h = pre.shape
    de = ef.shape[1]
    d_out = w3.shape[1]
    be = 4000
    grid = (e_tot // be,)
    b1r = b1.reshape(1, h)
    b2r = b2.reshape(1, h)
    b3r = b3.reshape(1, d_out)
    return pl.pallas_call(
        _mlp_body,
        grid=grid,
        in_specs=[
            pl.BlockSpec((be, h), lambda i: (i, 0)),
            pl.BlockSpec((be, de), lambda i: (i, 0)),
            pl.BlockSpec((de, h), lambda i: (0, 0)),
            pl.BlockSpec((1, h), lambda i: (0, 0)),
            pl.BlockSpec((h, h), lambda i: (0, 0)),
            pl.BlockSpec((1, h), lambda i: (0, 0)),
            pl.BlockSpec((h, d_out), lambda i: (0, 0)),
            pl.BlockSpec((1, d_out), lambda i: (0, 0)),
        ],
        out_specs=pl.BlockSpec((be, d_out), lambda i: (i, 0)),
        out_shape=jax.ShapeDtypeStruct((e_tot, d_out), jnp.float32),
    )(pre, ef, w1c, b1r, w2, b2r, w3, b3r)


def _sums_body(e_tot, npad, d, msg_hbm, tgt_hbm, sums_hbm,
               msg_v, idx_v, sum_tbl):
    c = lax.axis_index("c")
    s = lax.axis_index("s")
    wid = s * _NC + c
    ew = e_tot // _NW
    rpt = npad // _NS
    kper = rpt // _CHUNK

    def zrow(r, carry):
        for j in range(d // 16):
            msg_v[r, pl.ds(j * 16, 16)] = jnp.zeros((16,), jnp.float32)
        return carry

    lax.fori_loop(0, _CHUNK, zrow, 0)
    for k in range(kper):
        sl = pl.ds(s * rpt + k * _CHUNK, _CHUNK)
        pltpu.sync_copy(msg_v, sum_tbl.at[sl])
    plsc.subcore_barrier()

    def chunk(i, carry):
        base = wid * ew + i * _CHUNK
        pltpu.sync_copy(tgt_hbm.at[pl.ds(base, _CHUNK)], idx_v)
        pltpu.sync_copy(msg_hbm.at[pl.ds(base, _CHUNK)], msg_v)
        pltpu.sync_copy(msg_v, sum_tbl.at[idx_v], add=True)
        return carry

    lax.fori_loop(0, ew // _CHUNK, chunk, 0)
    plsc.subcore_barrier()

    for k in range(kper):
        sl = pl.ds(s * rpt + k * _CHUNK, _CHUNK)
        osl = pl.ds(c * npad + s * rpt + k * _CHUNK, _CHUNK)
        pltpu.sync_copy(sum_tbl.at[sl], msg_v)
        pltpu.sync_copy(msg_v, sums_hbm.at[osl])


def _scatter_sums(msg, tgt, npad):
    e_tot, d = msg.shape
    mesh = plsc.VectorSubcoreMesh(core_axis_name="c", subcore_axis_name="s")
    kern = pl.kernel(
        functools.partial(_sums_body, e_tot, npad, d),
        out_type=jax.ShapeDtypeStruct((_NC * npad, d), jnp.float32),
        mesh=mesh,
        scratch_types=[
            pltpu.VMEM((_CHUNK, d), jnp.float32),
            pltpu.VMEM((_CHUNK,), jnp.int32),
            pltpu.VMEM_SHARED((npad, d), jnp.float32),
        ],
    )
    return kern(msg, tgt)


def _cnts_body(e_tot, npad, d, tgt_hbm, cnts_hbm, ones_v, idx_v, cnt_tbl):
    c = lax.axis_index("c")
    s = lax.axis_index("s")
    wid = s * _NC + c
    ew = e_tot // _NW
    rpt = npad // _NS
    kper = rpt // _CHUNK
    one16 = jnp.full((16,), 1.0, jnp.float32)
    zero16 = jnp.zeros((16,), jnp.float32)

    def zrow(r, carry):
        for j in range(d // 16):
            ones_v[r, pl.ds(j * 16, 16)] = zero16
        return carry

    lax.fori_loop(0, _CHUNK, zrow, 0)
    for k in range(kper):
        sl = pl.ds(s * rpt + k * _CHUNK, _CHUNK)
        pltpu.sync_copy(ones_v, cnt_tbl.at[sl])

    def onerow(r, carry):
        for j in range(d // 16):
            ones_v[r, pl.ds(j * 16, 16)] = one16
        return carry

    lax.fori_loop(0, _CHUNK, onerow, 0)
    plsc.subcore_barrier()

    def chunk(i, carry):
        base = wid * ew + i * _CHUNK
        pltpu.sync_copy(tgt_hbm.at[pl.ds(base, _CHUNK)], idx_v)
        pltpu.sync_copy(ones_v, cnt_tbl.at[idx_v], add=True)
        return carry

    lax.fori_loop(0, ew // _CHUNK, chunk, 0)
    plsc.subcore_barrier()

    for k in range(kper):
        sl = pl.ds(s * rpt + k * _CHUNK, _CHUNK)
        osl = pl.ds(c * npad + s * rpt + k * _CHUNK, _CHUNK)
        pltpu.sync_copy(cnt_tbl.at[sl], ones_v)
        pltpu.sync_copy(ones_v, cnts_hbm.at[osl])


def _scatter_counts(tgt, npad, d):
    e_tot = tgt.shape[0]
    mesh = plsc.VectorSubcoreMesh(core_axis_name="c", subcore_axis_name="s")
    kern = pl.kernel(
        functools.partial(_cnts_body, e_tot, npad, d),
        out_type=jax.ShapeDtypeStruct((_NC * npad, d), jnp.float32),
        mesh=mesh,
        scratch_types=[
            pltpu.VMEM((_CHUNK, d), jnp.float32),
            pltpu.VMEM((_CHUNK,), jnp.int32),
            pltpu.VMEM_SHARED((npad, d), jnp.float32),
        ],
    )
    return kern(tgt)


def _final_body(x_ref, s0_ref, s1_ref, c0_ref, c1_ref, o_ref):
    ssum = s0_ref[...] + s1_ref[...]
    cnt = c0_ref[...] + c1_ref[...]
    o_ref[...] = x_ref[...] + ssum / jnp.maximum(cnt, 1.0)


def _finalize(x, sums, cnts, npad):
    n, d = x.shape
    bn = 2000
    grid = ((n + bn - 1) // bn,)
    s0, s1 = sums[:npad], sums[npad:]
    c0, c1 = cnts[:npad], cnts[npad:]
    return pl.pallas_call(
        _final_body,
        grid=grid,
        in_specs=[
            pl.BlockSpec((bn, d), lambda i: (i, 0)),
            pl.BlockSpec((bn, d), lambda i: (i, 0)),
            pl.BlockSpec((bn, d), lambda i: (i, 0)),
            pl.BlockSpec((bn, d), lambda i: (i, 0)),
            pl.BlockSpec((bn, d), lambda i: (i, 0)),
        ],
        out_specs=pl.BlockSpec((bn, d), lambda i: (i, 0)),
        out_shape=jax.ShapeDtypeStruct((n, d), jnp.float32),
    )(x, s0, s1, c0, c1)


def kernel(x, edge_index, edge_feat, W1, b1, W2, b2, W3, b3):
    n, d = x.shape
    src = edge_index[0]
    dst = edge_index[1]
    w_ab = W1[: 2 * d, :]
    w1c = W1[2 * d:, :]
    npad = _npad_for(n)

    xs, xt = _node_proj(x, w_ab)
    pre = _gather_add(xs, xt, src, dst)
    cnts = _scatter_counts(dst, npad, d)
    msg = _edge_mlp(pre, edge_feat, w1c, b1, W2, b2, W3, b3)
    sums = _scatter_sums(msg, dst, npad)
    return _finalize(x, sums, cnts, npad)


# speedup vs baseline: 5.2416x; 1.5985x over previous
import functools

import jax
import jax.numpy as jnp
from jax import lax
from jax.experimental import pallas as pl
from jax.experimental.pallas import tpu as pltpu
from jax.experimental.pallas import tpu_sc as plsc

_NC = 2
_NS = 16
_NW = _NC * _NS
_CHUNK = 80

_PREC = None


def _npad_for(n):
    blk = _NS * _CHUNK
    return ((n + blk - 1) // blk) * blk


def _proj_body(x_ref, w_ref, xs_ref, xt_ref):
    xb = x_ref[...]
    w = w_ref[...]
    d = xb.shape[1]
    xs_ref[...] = jnp.dot(xb, w[:d, :], precision=_PREC,
                          preferred_element_type=jnp.float32)
    xt_ref[...] = jnp.dot(xb, w[d:, :], precision=_PREC,
                          preferred_element_type=jnp.float32)


def _node_proj(x, w_ab):
    n, d = x.shape
    h = w_ab.shape[1]
    bn = 2000
    grid = (n // bn,)
    return pl.pallas_call(
        _proj_body,
        grid=grid,
        in_specs=[
            pl.BlockSpec((bn, d), lambda i: (i, 0)),
            pl.BlockSpec((2 * d, h), lambda i: (0, 0)),
        ],
        out_specs=[
            pl.BlockSpec((bn, h), lambda i: (i, 0)),
            pl.BlockSpec((bn, h), lambda i: (i, 0)),
        ],
        out_shape=[
            jax.ShapeDtypeStruct((n, h), jnp.float32),
            jax.ShapeDtypeStruct((n, h), jnp.float32),
        ],
    )(x, w_ab)


def _gather_body(e_tot, h, xs_hbm, xt_hbm, src3_hbm, dst3_hbm, out_hbm,
                 idx_s, idx_t, ra0, rb0, ob0, ra1, rb1, ob1,
                 sem0, sem1, semo0, semo1):
    c = lax.axis_index("c")
    s = lax.axis_index("s")
    wid = s * _NC + c
    ew = e_tot // _NW
    nchunk = ew // _CHUNK
    nvec = h // 16
    pltpu.sync_copy(src3_hbm.at[wid], idx_s)
    pltpu.sync_copy(dst3_hbm.at[wid], idx_t)
    slots = ((ra0, rb0, ob0, sem0, semo0), (ra1, rb1, ob1, sem1, semo1))

    def issue(k, slot):
        ra, rb, ob, sem, semo = slot
        pltpu.async_copy(xs_hbm.at[idx_s.at[k]], ra, sem)
        pltpu.async_copy(xt_hbm.at[idx_t.at[k]], rb, sem)

    def consume(k, slot):
        ra, rb, ob, sem, semo = slot
        pltpu.make_async_copy(xs_hbm.at[idx_s.at[k]], ra, sem).wait()
        pltpu.make_async_copy(xt_hbm.at[idx_t.at[k]], rb, sem).wait()
        base = wid * ew + k * _CHUNK

        @pl.when(k >= 2)
        def _():
            pltpu.make_async_copy(ob, out_hbm.at[pl.ds(base, _CHUNK)],
                                  semo).wait()

        def row(r, rc):
            for j in range(nvec):
                sl = pl.ds(j * 16, 16)
                ob[r, sl] = ra[r, sl] + rb[r, sl]
            return rc

        lax.fori_loop(0, _CHUNK, row, 0)
        pltpu.async_copy(ob, out_hbm.at[pl.ds(base, _CHUNK)], semo)

    issue(0, slots[0])
    issue(1, slots[1])

    def pair(g, carry):
        for b in range(2):
            k = g * 2 + b
            consume(k, slots[b])
            nk = k + 2

            @pl.when(nk < nchunk)
            def _():
                issue(nk, slots[b])
        return carry

    lax.fori_loop(0, nchunk // 2, pair, 0)
    if nchunk % 2 == 1:
        consume(nchunk - 1, slots[0])
    tail = wid * ew
    for b in range(2):
        pltpu.make_async_copy(slots[b][2], out_hbm.at[pl.ds(tail, _CHUNK)],
                              slots[b][4]).wait()


def _gather_add(xs, xt, src3, dst3):
    n, h = xs.shape
    e_tot = src3.shape[0] * src3.shape[1] * src3.shape[2]
    nchunk = src3.shape[1]
    mesh = plsc.VectorSubcoreMesh(core_axis_name="c", subcore_axis_name="s")
    kern = pl.kernel(
        functools.partial(_gather_body, e_tot, h),
        out_type=jax.ShapeDtypeStruct((e_tot, h), jnp.float32),
        mesh=mesh,
        scratch_types=[
            pltpu.VMEM((nchunk, _CHUNK), jnp.int32),
            pltpu.VMEM((nchunk, _CHUNK), jnp.int32),
            pltpu.VMEM((_CHUNK, h), jnp.float32),
            pltpu.VMEM((_CHUNK, h), jnp.float32),
            pltpu.VMEM((_CHUNK, h), jnp.float32),
            pltpu.VMEM((_CHUNK, h), jnp.float32),
            pltpu.VMEM((_CHUNK, h), jnp.float32),
            pltpu.VMEM((_CHUNK, h), jnp.float32),
            pltpu.SemaphoreType.DMA,
            pltpu.SemaphoreType.DMA,
            pltpu.SemaphoreType.DMA,
            pltpu.SemaphoreType.DMA,
        ],
    )
    return kern(xs, xt, src3, dst3)


def _mlp_body(pre_ref, ef_ref, w1c_ref, b1_ref, w2_ref, b2_ref, w3_ref, b3_ref,
              msg_ref):
    pre = pre_ref[...]
    ef = ef_ref[...]
    h1 = pre + jnp.dot(ef, w1c_ref[...], precision=_PREC,
                       preferred_element_type=jnp.float32) + b1_ref[...]
    h1 = jnp.maximum(h1, 0.0)
    h2 = jnp.dot(h1, w2_ref[...], precision=_PREC,
                 preferred_element_type=jnp.float32) + b2_ref[...]
    h2 = jnp.maximum(h2, 0.0)
    msg_ref[...] = jnp.dot(h2, w3_ref[...], precision=_PREC,
                           preferred_element_type=jnp.float32) + b3_ref[...]


def _edge_mlp(pre, ef, w1c, b1, w2, b2, w3, b3):
    e_tot, h = pre.shape
    de = ef.shape[1]
    d_out = w3.shape[1]
    be = 4000
    grid = (e_tot // be,)
    b1r = b1.reshape(1, h)
    b2r = b2.reshape(1, h)
    b3r = b3.reshape(1, d_out)
    return pl.pallas_call(
        _mlp_body,
        grid=grid,
        in_specs=[
            pl.BlockSpec((be, h), lambda i: (i, 0)),
            pl.BlockSpec((be, de), lambda i: (i, 0)),
            pl.BlockSpec((de, h), lambda i: (0, 0)),
            pl.BlockSpec((1, h), lambda i: (0, 0)),
            pl.BlockSpec((h, h), lambda i: (0, 0)),
            pl.BlockSpec((1, h), lambda i: (0, 0)),
            pl.BlockSpec((h, d_out), lambda i: (0, 0)),
            pl.BlockSpec((1, d_out), lambda i: (0, 0)),
        ],
        out_specs=pl.BlockSpec((be, d_out), lambda i: (i, 0)),
        out_shape=jax.ShapeDtypeStruct((e_tot, d_out), jnp.float32),
    )(pre, ef, w1c, b1r, w2, b2r, w3, b3r)


def _sums_body(e_tot, npad, d, msg_hbm, tgt3_hbm, sums_hbm,
               msg_v, msg_v1, idx_all, sum_tbl, semm0, semm1):
    c = lax.axis_index("c")
    s = lax.axis_index("s")
    wid = s * _NC + c
    ew = e_tot // _NW
    rpt = npad // _NS
    kper = rpt // _CHUNK

    def zrow(r, carry):
        for j in range(d // 16):
            msg_v[r, pl.ds(j * 16, 16)] = jnp.zeros((16,), jnp.float32)
        return carry

    lax.fori_loop(0, _CHUNK, zrow, 0)
    for k in range(kper):
        sl = pl.ds(s * rpt + k * _CHUNK, _CHUNK)
        pltpu.sync_copy(msg_v, sum_tbl.at[sl])
    plsc.subcore_barrier()

    pltpu.sync_copy(tgt3_hbm.at[wid], idx_all)
    nchunk = ew // _CHUNK
    slots = ((msg_v, semm0), (msg_v1, semm1))

    def issue(k, slot):
        mv, sem = slot
        base = wid * ew + k * _CHUNK
        pltpu.async_copy(msg_hbm.at[pl.ds(base, _CHUNK)], mv, sem)

    def consume(k, slot):
        mv, sem = slot
        base = wid * ew + k * _CHUNK
        pltpu.make_async_copy(msg_hbm.at[pl.ds(base, _CHUNK)], mv, sem).wait()
        pltpu.sync_copy(mv, sum_tbl.at[idx_all.at[k]], add=True)

    issue(0, slots[0])
    issue(1, slots[1])

    def pair(g, carry):
        for b in range(2):
            k = g * 2 + b
            consume(k, slots[b])
            nk = k + 2

            @pl.when(nk < nchunk)
            def _():
                issue(nk, slots[b])
        return carry

    lax.fori_loop(0, nchunk // 2, pair, 0)
    if nchunk % 2 == 1:
        consume(nchunk - 1, slots[0])
    plsc.subcore_barrier()

    for k in range(kper):
        sl = pl.ds(s * rpt + k * _CHUNK, _CHUNK)
        osl = pl.ds(c * npad + s * rpt + k * _CHUNK, _CHUNK)
        pltpu.sync_copy(sum_tbl.at[sl], msg_v)
        pltpu.sync_copy(msg_v, sums_hbm.at[osl])


def _scatter_sums(msg, tgt3, npad):
    e_tot, d = msg.shape
    nchunk = tgt3.shape[1]
    mesh = plsc.VectorSubcoreMesh(core_axis_name="c", subcore_axis_name="s")
    kern = pl.kernel(
        functools.partial(_sums_body, e_tot, npad, d),
        out_type=jax.ShapeDtypeStruct((_NC * npad, d), jnp.float32),
        mesh=mesh,
        scratch_types=[
            pltpu.VMEM((_CHUNK, d), jnp.float32),
            pltpu.VMEM((_CHUNK, d), jnp.float32),
            pltpu.VMEM((nchunk, _CHUNK), jnp.int32),
            pltpu.VMEM_SHARED((npad, d), jnp.float32),
            pltpu.SemaphoreType.DMA,
            pltpu.SemaphoreType.DMA,
        ],
    )
    return kern(msg, tgt3)


def _cnts_body(e_tot, npad, d, tgt_hbm, cnts_hbm, ones_v, idx_v, cnt_tbl):
    c = lax.axis_index("c")
    s = lax.axis_index("s")
    wid = s * _NC + c
    ew = e_tot // _NW
    rpt = npad // _NS
    kper = rpt // _CHUNK
    one16 = jnp.full((16,), 1.0, jnp.float32)
    zero16 = jnp.zeros((16,), jnp.float32)

    def zrow(r, carry):
        for j in range(d // 16):
            ones_v[r, pl.ds(j * 16, 16)] = zero16
        return carry

    lax.fori_loop(0, _CHUNK, zrow, 0)
    for k in range(kper):
        sl = pl.ds(s * rpt + k * _CHUNK, _CHUNK)
        pltpu.sync_copy(ones_v, cnt_tbl.at[sl])

    def onerow(r, carry):
        for j in range(d // 16):
            ones_v[r, pl.ds(j * 16, 16)] = one16
        return carry

    lax.fori_loop(0, _CHUNK, onerow, 0)
    plsc.subcore_barrier()

    def chunk(i, carry):
        base = wid * ew + i * _CHUNK
        pltpu.sync_copy(tgt_hbm.at[pl.ds(base, _CHUNK)], idx_v)
        pltpu.sync_copy(ones_v, cnt_tbl.at[idx_v], add=True)
        return carry

    lax.fori_loop(0, ew // _CHUNK, chunk, 0)
    plsc.subcore_barrier()

    for k in range(kper):
        sl = pl.ds(s * rpt + k * _CHUNK, _CHUNK)
        osl = pl.ds(c * npad + s * rpt + k * _CHUNK, _CHUNK)
        pltpu.sync_copy(cnt_tbl.at[sl], ones_v)
        pltpu.sync_copy(ones_v, cnts_hbm.at[osl])


def _scatter_counts(tgt, npad, d):
    e_tot = tgt.shape[0]
    mesh = plsc.VectorSubcoreMesh(core_axis_name="c", subcore_axis_name="s")
    kern = pl.kernel(
        functools.partial(_cnts_body, e_tot, npad, d),
        out_type=jax.ShapeDtypeStruct((_NC * npad, d), jnp.float32),
        mesh=mesh,
        scratch_types=[
            pltpu.VMEM((_CHUNK, d), jnp.float32),
            pltpu.VMEM((_CHUNK,), jnp.int32),
            pltpu.VMEM_SHARED((npad, d), jnp.float32),
        ],
    )
    return kern(tgt)


def _final_body(x_ref, s0_ref, s1_ref, c0_ref, c1_ref, o_ref):
    ssum = s0_ref[...] + s1_ref[...]
    cnt = c0_ref[...] + c1_ref[...]
    o_ref[...] = x_ref[...] + ssum / jnp.maximum(cnt, 1.0)


def _finalize(x, sums, cnts, npad):
    n, d = x.shape
    bn = 2000
    grid = ((n + bn - 1) // bn,)
    s0, s1 = sums[:npad], sums[npad:]
    c0, c1 = cnts[:npad], cnts[npad:]
    return pl.pallas_call(
        _final_body,
        grid=grid,
        in_specs=[
            pl.BlockSpec((bn, d), lambda i: (i, 0)),
            pl.BlockSpec((bn, d), lambda i: (i, 0)),
            pl.BlockSpec((bn, d), lambda i: (i, 0)),
            pl.BlockSpec((bn, d), lambda i: (i, 0)),
            pl.BlockSpec((bn, d), lambda i: (i, 0)),
        ],
        out_specs=pl.BlockSpec((bn, d), lambda i: (i, 0)),
        out_shape=jax.ShapeDtypeStruct((n, d), jnp.float32),
    )(x, s0, s1, c0, c1)


def kernel(x, edge_index, edge_feat, W1, b1, W2, b2, W3, b3):
    n, d = x.shape
    src = edge_index[0]
    dst = edge_index[1]
    w_ab = W1[: 2 * d, :]
    w1c = W1[2 * d:, :]
    npad = _npad_for(n)

    e_tot = src.shape[0]
    ew = e_tot // _NW
    src3 = src.reshape(_NW, ew // _CHUNK, _CHUNK)
    dst3 = dst.reshape(_NW, ew // _CHUNK, _CHUNK)

    xs, xt = _node_proj(x, w_ab)
    pre = _gather_add(xs, xt, src3, dst3)
    cnts = _scatter_counts(dst, npad, d)
    msg = _edge_mlp(pre, edge_feat, w1c, b1, W2, b2, W3, b3)
    sums = _scatter_sums(msg, dst3, npad)
    return _finalize(x, sums, cnts, npad)


# speedup vs baseline: 5.2567x; 1.0029x over previous
import functools

import jax
import jax.numpy as jnp
from jax import lax
from jax.experimental import pallas as pl
from jax.experimental.pallas import tpu as pltpu
from jax.experimental.pallas import tpu_sc as plsc

_NC = 2
_NS = 16
_NW = _NC * _NS
_CHUNK = 80

_PREC = None


def _npad_for(n):
    blk = _NS * _CHUNK
    return ((n + blk - 1) // blk) * blk


def _proj_body(x_ref, w_ref, xs_ref, xt_ref):
    xb = x_ref[...]
    w = w_ref[...]
    d = xb.shape[1]
    xs_ref[...] = jnp.dot(xb, w[:d, :], precision=_PREC,
                          preferred_element_type=jnp.float32)
    xt_ref[...] = jnp.dot(xb, w[d:, :], precision=_PREC,
                          preferred_element_type=jnp.float32)


def _node_proj(x, w_ab):
    n, d = x.shape
    h = w_ab.shape[1]
    bn = 2000
    grid = (n // bn,)
    return pl.pallas_call(
        _proj_body,
        grid=grid,
        in_specs=[
            pl.BlockSpec((bn, d), lambda i: (i, 0)),
            pl.BlockSpec((2 * d, h), lambda i: (0, 0)),
        ],
        out_specs=[
            pl.BlockSpec((bn, h), lambda i: (i, 0)),
            pl.BlockSpec((bn, h), lambda i: (i, 0)),
        ],
        out_shape=[
            jax.ShapeDtypeStruct((n, h), jnp.float32),
            jax.ShapeDtypeStruct((n, h), jnp.float32),
        ],
    )(x, w_ab)


def _gather_body(e_tot, h, xs_hbm, xt_hbm, src3_hbm, dst3_hbm, out_hbm,
                 idx_s, idx_t, ra0, rb0, ob0, ra1, rb1, ob1,
                 sem0, sem1, semo0, semo1):
    c = lax.axis_index("c")
    s = lax.axis_index("s")
    wid = s * _NC + c
    ew = e_tot // _NW
    nchunk = ew // _CHUNK
    nvec = h // 16
    pltpu.sync_copy(src3_hbm.at[wid], idx_s)
    pltpu.sync_copy(dst3_hbm.at[wid], idx_t)
    slots = ((ra0, rb0, ob0, sem0, semo0), (ra1, rb1, ob1, sem1, semo1))

    def issue(k, slot):
        ra, rb, ob, sem, semo = slot
        pltpu.async_copy(xs_hbm.at[idx_s.at[k]], ra, sem)
        pltpu.async_copy(xt_hbm.at[idx_t.at[k]], rb, sem)

    def consume(k, slot):
        ra, rb, ob, sem, semo = slot
        pltpu.make_async_copy(xs_hbm.at[idx_s.at[k]], ra, sem).wait()
        pltpu.make_async_copy(xt_hbm.at[idx_t.at[k]], rb, sem).wait()
        base = wid * ew + k * _CHUNK

        @pl.when(k >= 2)
        def _():
            pltpu.make_async_copy(ob, out_hbm.at[pl.ds(base, _CHUNK)],
                                  semo).wait()

        def row(r, rc):
            for j in range(nvec):
                sl = pl.ds(j * 16, 16)
                ob[r, sl] = ra[r, sl] + rb[r, sl]
            return rc

        lax.fori_loop(0, _CHUNK, row, 0)
        pltpu.async_copy(ob, out_hbm.at[pl.ds(base, _CHUNK)], semo)

    issue(0, slots[0])
    issue(1, slots[1])

    def pair(g, carry):
        for b in range(2):
            k = g * 2 + b
            consume(k, slots[b])
            nk = k + 2

            @pl.when(nk < nchunk)
            def _():
                issue(nk, slots[b])
        return carry

    lax.fori_loop(0, nchunk // 2, pair, 0)
    if nchunk % 2 == 1:
        consume(nchunk - 1, slots[0])
    tail = wid * ew
    for b in range(2):
        pltpu.make_async_copy(slots[b][2], out_hbm.at[pl.ds(tail, _CHUNK)],
                              slots[b][4]).wait()


def _gather_add(xs, xt, src3, dst3):
    n, h = xs.shape
    e_tot = src3.shape[0] * src3.shape[1] * src3.shape[2]
    nchunk = src3.shape[1]
    mesh = plsc.VectorSubcoreMesh(core_axis_name="c", subcore_axis_name="s")
    kern = pl.kernel(
        functools.partial(_gather_body, e_tot, h),
        out_type=jax.ShapeDtypeStruct((e_tot, h), jnp.float32),
        mesh=mesh,
        scratch_types=[
            pltpu.VMEM((nchunk, _CHUNK), jnp.int32),
            pltpu.VMEM((nchunk, _CHUNK), jnp.int32),
            pltpu.VMEM((_CHUNK, h), jnp.float32),
            pltpu.VMEM((_CHUNK, h), jnp.float32),
            pltpu.VMEM((_CHUNK, h), jnp.float32),
            pltpu.VMEM((_CHUNK, h), jnp.float32),
            pltpu.VMEM((_CHUNK, h), jnp.float32),
            pltpu.VMEM((_CHUNK, h), jnp.float32),
            pltpu.SemaphoreType.DMA,
            pltpu.SemaphoreType.DMA,
            pltpu.SemaphoreType.DMA,
            pltpu.SemaphoreType.DMA,
        ],
    )
    return kern(xs, xt, src3, dst3)


def _mlp_body(pre_ref, ef_ref, w1c_ref, b1_ref, w2_ref, b2_ref, w3_ref, b3_ref,
              msg_ref):
    pre = pre_ref[...]
    ef = ef_ref[...]
    h1 = pre + jnp.dot(ef, w1c_ref[...], precision=_PREC,
                       preferred_element_type=jnp.float32) + b1_ref[...]
    h1 = jnp.maximum(h1, 0.0)
    h2 = jnp.dot(h1, w2_ref[...], precision=_PREC,
                 preferred_element_type=jnp.float32) + b2_ref[...]
    h2 = jnp.maximum(h2, 0.0)
    msg_ref[...] = jnp.dot(h2, w3_ref[...], precision=_PREC,
                           preferred_element_type=jnp.float32) + b3_ref[...]


def _edge_mlp(pre, ef, w1c, b1, w2, b2, w3, b3):
    e_tot, h = pre.shape
    de = ef.shape[1]
    d_out = w3.shape[1]
    be = 4000
    grid = (e_tot // be,)
    b1r = b1.reshape(1, h)
    b2r = b2.reshape(1, h)
    b3r = b3.reshape(1, d_out)
    return pl.pallas_call(
        _mlp_body,
        grid=grid,
        in_specs=[
            pl.BlockSpec((be, h), lambda i: (i, 0)),
            pl.BlockSpec((be, de), lambda i: (i, 0)),
            pl.BlockSpec((de, h), lambda i: (0, 0)),
            pl.BlockSpec((1, h), lambda i: (0, 0)),
            pl.BlockSpec((h, h), lambda i: (0, 0)),
            pl.BlockSpec((1, h), lambda i: (0, 0)),
            pl.BlockSpec((h, d_out), lambda i: (0, 0)),
            pl.BlockSpec((1, d_out), lambda i: (0, 0)),
        ],
        out_specs=pl.BlockSpec((be, d_out), lambda i: (i, 0)),
        out_shape=jax.ShapeDtypeStruct((e_tot, d_out), jnp.float32),
    )(pre, ef, w1c, b1r, w2, b2r, w3, b3r)


def _sums_body(e_tot, npad, d, msg_hbm, tgt3_hbm, sums_hbm,
               msg_v, msg_v1, idx_all, sum_tbl, semm0, semm1):
    c = lax.axis_index("c")
    s = lax.axis_index("s")
    wid = s * _NC + c
    ew = e_tot // _NW
    rpt = npad // _NS
    kper = rpt // _CHUNK

    def zrow(r, carry):
        for j in range(d // 16):
            msg_v[r, pl.ds(j * 16, 16)] = jnp.zeros((16,), jnp.float32)
        return carry

    lax.fori_loop(0, _CHUNK, zrow, 0)
    for k in range(kper):
        sl = pl.ds(s * rpt + k * _CHUNK, _CHUNK)
        pltpu.sync_copy(msg_v, sum_tbl.at[sl])
    plsc.subcore_barrier()

    pltpu.sync_copy(tgt3_hbm.at[wid], idx_all)
    nchunk = ew // _CHUNK
    slots = ((msg_v, semm0), (msg_v1, semm1))

    def issue(k, slot):
        mv, sem = slot
        base = wid * ew + k * _CHUNK
        pltpu.async_copy(msg_hbm.at[pl.ds(base, _CHUNK)], mv, sem)

    def consume(k, slot):
        mv, sem = slot
        base = wid * ew + k * _CHUNK
        pltpu.make_async_copy(msg_hbm.at[pl.ds(base, _CHUNK)], mv, sem).wait()
        pltpu.sync_copy(mv, sum_tbl.at[idx_all.at[k]], add=True)

    issue(0, slots[0])
    issue(1, slots[1])

    def pair(g, carry):
        for b in range(2):
            k = g * 2 + b
            consume(k, slots[b])
            nk = k + 2

            @pl.when(nk < nchunk)
            def _():
                issue(nk, slots[b])
        return carry

    lax.fori_loop(0, nchunk // 2, pair, 0)
    if nchunk % 2 == 1:
        consume(nchunk - 1, slots[0])
    plsc.subcore_barrier()

    for k in range(kper):
        sl = pl.ds(s * rpt + k * _CHUNK, _CHUNK)
        osl = pl.ds(c * npad + s * rpt + k * _CHUNK, _CHUNK)
        pltpu.sync_copy(sum_tbl.at[sl], msg_v)
        pltpu.sync_copy(msg_v, sums_hbm.at[osl])


def _scatter_sums(msg, tgt3, npad):
    e_tot, d = msg.shape
    nchunk = tgt3.shape[1]
    mesh = plsc.VectorSubcoreMesh(core_axis_name="c", subcore_axis_name="s")
    kern = pl.kernel(
        functools.partial(_sums_body, e_tot, npad, d),
        out_type=jax.ShapeDtypeStruct((_NC * npad, d), jnp.float32),
        mesh=mesh,
        scratch_types=[
            pltpu.VMEM((_CHUNK, d), jnp.float32),
            pltpu.VMEM((_CHUNK, d), jnp.float32),
            pltpu.VMEM((nchunk, _CHUNK), jnp.int32),
            pltpu.VMEM_SHARED((npad, d), jnp.float32),
            pltpu.SemaphoreType.DMA,
            pltpu.SemaphoreType.DMA,
        ],
    )
    return kern(msg, tgt3)


def _cnts_body(e_tot, npad, d, tgt3_hbm, cnts_hbm, ones_v, idx_all, cnt_tbl,
               sem):
    c = lax.axis_index("c")
    s = lax.axis_index("s")
    wid = s * _NC + c
    ew = e_tot // _NW
    rpt = npad // _NS
    kper = rpt // _CHUNK
    one16 = jnp.full((16,), 1.0, jnp.float32)
    zero16 = jnp.zeros((16,), jnp.float32)

    def zrow(r, carry):
        for j in range(d // 16):
            ones_v[r, pl.ds(j * 16, 16)] = zero16
        return carry

    lax.fori_loop(0, _CHUNK, zrow, 0)
    for k in range(kper):
        sl = pl.ds(s * rpt + k * _CHUNK, _CHUNK)
        pltpu.sync_copy(ones_v, cnt_tbl.at[sl])

    def onerow(r, carry):
        for j in range(d // 16):
            ones_v[r, pl.ds(j * 16, 16)] = one16
        return carry

    lax.fori_loop(0, _CHUNK, onerow, 0)
    plsc.subcore_barrier()

    pltpu.sync_copy(tgt3_hbm.at[wid], idx_all)
    nchunk = ew // _CHUNK
    grp = 5

    def group_fn(g, carry):
        for b in range(grp):
            k = g * grp + b
            pltpu.async_copy(ones_v, cnt_tbl.at[idx_all.at[k]], sem,
                             add=True)
        for b in range(grp):
            k = g * grp + b
            pltpu.make_async_copy(ones_v, cnt_tbl.at[idx_all.at[k]],
                                  sem).wait()
        return carry

    lax.fori_loop(0, nchunk // grp, group_fn, 0)
    plsc.subcore_barrier()

    for k in range(kper):
        sl = pl.ds(s * rpt + k * _CHUNK, _CHUNK)
        osl = pl.ds(c * npad + s * rpt + k * _CHUNK, _CHUNK)
        pltpu.sync_copy(cnt_tbl.at[sl], ones_v)
        pltpu.sync_copy(ones_v, cnts_hbm.at[osl])


def _scatter_counts(tgt3, npad, d):
    e_tot = tgt3.shape[0] * tgt3.shape[1] * tgt3.shape[2]
    nchunk = tgt3.shape[1]
    mesh = plsc.VectorSubcoreMesh(core_axis_name="c", subcore_axis_name="s")
    kern = pl.kernel(
        functools.partial(_cnts_body, e_tot, npad, d),
        out_type=jax.ShapeDtypeStruct((_NC * npad, d), jnp.float32),
        mesh=mesh,
        scratch_types=[
            pltpu.VMEM((_CHUNK, d), jnp.float32),
            pltpu.VMEM((nchunk, _CHUNK), jnp.int32),
            pltpu.VMEM_SHARED((npad, d), jnp.float32),
            pltpu.SemaphoreType.DMA,
        ],
    )
    return kern(tgt3)


def _final_body(x_ref, s0_ref, s1_ref, c0_ref, c1_ref, o_ref):
    ssum = s0_ref[...] + s1_ref[...]
    cnt = c0_ref[...] + c1_ref[...]
    o_ref[...] = x_ref[...] + ssum / jnp.maximum(cnt, 1.0)


def _finalize(x, sums, cnts, npad):
    n, d = x.shape
    bn = 2000
    grid = ((n + bn - 1) // bn,)
    s0, s1 = sums[:npad], sums[npad:]
    c0, c1 = cnts[:npad], cnts[npad:]
    return pl.pallas_call(
        _final_body,
        grid=grid,
        in_specs=[
            pl.BlockSpec((bn, d), lambda i: (i, 0)),
            pl.BlockSpec((bn, d), lambda i: (i, 0)),
            pl.BlockSpec((bn, d), lambda i: (i, 0)),
            pl.BlockSpec((bn, d), lambda i: (i, 0)),
            pl.BlockSpec((bn, d), lambda i: (i, 0)),
        ],
        out_specs=pl.BlockSpec((bn, d), lambda i: (i, 0)),
        out_shape=jax.ShapeDtypeStruct((n, d), jnp.float32),
    )(x, s0, s1, c0, c1)


def kernel(x, edge_index, edge_feat, W1, b1, W2, b2, W3, b3):
    n, d = x.shape
    src = edge_index[0]
    dst = edge_index[1]
    w_ab = W1[: 2 * d, :]
    w1c = W1[2 * d:, :]
    npad = _npad_for(n)

    e_tot = src.shape[0]
    ew = e_tot // _NW
    src3 = src.reshape(_NW, ew // _CHUNK, _CHUNK)
    dst3 = dst.reshape(_NW, ew // _CHUNK, _CHUNK)

    xs, xt = _node_proj(x, w_ab)
    pre = _gather_add(xs, xt, src3, dst3)
    cnts = _scatter_counts(dst3, npad, d)
    msg = _edge_mlp(pre, edge_feat, w1c, b1, W2, b2, W3, b3)
    sums = _scatter_sums(msg, dst3, npad)
    return _finalize(x, sums, cnts, npad)


# speedup vs baseline: 5.3701x; 1.0216x over previous
import functools

import jax
import jax.numpy as jnp
from jax import lax
from jax.experimental import pallas as pl
from jax.experimental.pallas import tpu as pltpu
from jax.experimental.pallas import tpu_sc as plsc

_NC = 2
_NS = 16
_NW = _NC * _NS
_CHUNK = 80

_PREC = None


def _npad_for(n):
    blk = _NS * _CHUNK
    return ((n + blk - 1) // blk) * blk


def _proj_body(x_ref, w_ref, xs_ref, xt_ref):
    xb = x_ref[...]
    w = w_ref[...]
    d = xb.shape[1]
    xs_ref[...] = jnp.dot(xb, w[:d, :], precision=_PREC,
                          preferred_element_type=jnp.float32)
    xt_ref[...] = jnp.dot(xb, w[d:, :], precision=_PREC,
                          preferred_element_type=jnp.float32)


def _node_proj(x, w_ab):
    n, d = x.shape
    h = w_ab.shape[1]
    bn = 2000
    grid = (n // bn,)
    return pl.pallas_call(
        _proj_body,
        grid=grid,
        in_specs=[
            pl.BlockSpec((bn, d), lambda i: (i, 0)),
            pl.BlockSpec((2 * d, h), lambda i: (0, 0)),
        ],
        out_specs=[
            pl.BlockSpec((bn, h), lambda i: (i, 0)),
            pl.BlockSpec((bn, h), lambda i: (i, 0)),
        ],
        out_shape=[
            jax.ShapeDtypeStruct((n, h), jnp.float32),
            jax.ShapeDtypeStruct((n, h), jnp.float32),
        ],
    )(x, w_ab)


def _gather_body(e_tot, h, xs_hbm, xt_hbm, src3_hbm, dst3_hbm, out_hbm,
                 idx_s, idx_t, ra0, rb0, ob0, ra1, rb1, ob1,
                 sem0, sem1, semo0, semo1):
    c = lax.axis_index("c")
    s = lax.axis_index("s")
    wid = s * _NC + c
    ew = e_tot // _NW
    nchunk = ew // _CHUNK
    nvec = h // 16
    pltpu.sync_copy(src3_hbm.at[wid], idx_s)
    pltpu.sync_copy(dst3_hbm.at[wid], idx_t)
    slots = ((ra0, rb0, ob0, sem0, semo0), (ra1, rb1, ob1, sem1, semo1))

    def issue(k, slot):
        ra, rb, ob, sem, semo = slot
        pltpu.async_copy(xs_hbm.at[idx_s.at[k]], ra, sem)
        pltpu.async_copy(xt_hbm.at[idx_t.at[k]], rb, sem)

    def consume(k, slot):
        ra, rb, ob, sem, semo = slot
        pltpu.make_async_copy(xs_hbm.at[idx_s.at[k]], ra, sem).wait()
        pltpu.make_async_copy(xt_hbm.at[idx_t.at[k]], rb, sem).wait()
        base = wid * ew + k * _CHUNK

        @pl.when(k >= 2)
        def _():
            pltpu.make_async_copy(ob, out_hbm.at[pl.ds(base, _CHUNK)],
                                  semo).wait()

        def row(r, rc):
            for j in range(nvec):
                sl = pl.ds(j * 16, 16)
                ob[r, sl] = ra[r, sl] + rb[r, sl]
            return rc

        lax.fori_loop(0, _CHUNK, row, 0)
        pltpu.async_copy(ob, out_hbm.at[pl.ds(base, _CHUNK)], semo)

    issue(0, slots[0])
    issue(1, slots[1])

    def pair(g, carry):
        for b in range(2):
            k = g * 2 + b
            consume(k, slots[b])
            nk = k + 2

            @pl.when(nk < nchunk)
            def _():
                issue(nk, slots[b])
        return carry

    lax.fori_loop(0, nchunk // 2, pair, 0)
    if nchunk % 2 == 1:
        consume(nchunk - 1, slots[0])
    tail = wid * ew
    for b in range(2):
        pltpu.make_async_copy(slots[b][2], out_hbm.at[pl.ds(tail, _CHUNK)],
                              slots[b][4]).wait()


def _gather_add(xs, xt, src3, dst3):
    n, h = xs.shape
    e_tot = src3.shape[0] * src3.shape[1] * src3.shape[2]
    nchunk = src3.shape[1]
    mesh = plsc.VectorSubcoreMesh(core_axis_name="c", subcore_axis_name="s")
    kern = pl.kernel(
        functools.partial(_gather_body, e_tot, h),
        out_type=jax.ShapeDtypeStruct((e_tot, h), jnp.float32),
        mesh=mesh,
        scratch_types=[
            pltpu.VMEM((nchunk, _CHUNK), jnp.int32),
            pltpu.VMEM((nchunk, _CHUNK), jnp.int32),
            pltpu.VMEM((_CHUNK, h), jnp.float32),
            pltpu.VMEM((_CHUNK, h), jnp.float32),
            pltpu.VMEM((_CHUNK, h), jnp.float32),
            pltpu.VMEM((_CHUNK, h), jnp.float32),
            pltpu.VMEM((_CHUNK, h), jnp.float32),
            pltpu.VMEM((_CHUNK, h), jnp.float32),
            pltpu.SemaphoreType.DMA,
            pltpu.SemaphoreType.DMA,
            pltpu.SemaphoreType.DMA,
            pltpu.SemaphoreType.DMA,
        ],
    )
    return kern(xs, xt, src3, dst3)


def _mlp_body(pre_ref, ef_ref, w1c_ref, b1_ref, w2_ref, b2_ref, w3_ref, b3_ref,
              msg_ref):
    pre = pre_ref[...]
    ef = ef_ref[...]
    h1 = pre + jnp.dot(ef, w1c_ref[...], precision=_PREC,
                       preferred_element_type=jnp.float32) + b1_ref[...]
    h1 = jnp.maximum(h1, 0.0)
    h2 = jnp.dot(h1, w2_ref[...], precision=_PREC,
                 preferred_element_type=jnp.float32) + b2_ref[...]
    h2 = jnp.maximum(h2, 0.0)
    msg_ref[...] = jnp.dot(h2, w3_ref[...], precision=_PREC,
                           preferred_element_type=jnp.float32) + b3_ref[...]


def _edge_mlp(pre, ef, w1c, b1, w2, b2, w3, b3):
    e_tot, h = pre.shape
    de = ef.shape[1]
    d_out = w3.shape[1]
    be = 8000
    grid = (e_tot // be,)
    b1r = b1.reshape(1, h)
    b2r = b2.reshape(1, h)
    b3r = b3.reshape(1, d_out)
    return pl.pallas_call(
        _mlp_body,
        grid=grid,
        in_specs=[
            pl.BlockSpec((be, h), lambda i: (i, 0)),
            pl.BlockSpec((be, de), lambda i: (i, 0)),
            pl.BlockSpec((de, h), lambda i: (0, 0)),
            pl.BlockSpec((1, h), lambda i: (0, 0)),
            pl.BlockSpec((h, h), lambda i: (0, 0)),
            pl.BlockSpec((1, h), lambda i: (0, 0)),
            pl.BlockSpec((h, d_out), lambda i: (0, 0)),
            pl.BlockSpec((1, d_out), lambda i: (0, 0)),
        ],
        out_specs=pl.BlockSpec((be, d_out), lambda i: (i, 0)),
        out_shape=jax.ShapeDtypeStruct((e_tot, d_out), jnp.float32),
    )(pre, ef, w1c, b1r, w2, b2r, w3, b3r)


def _sums_body(e_tot, npad, d, msg_hbm, tgt3_hbm, sums_hbm,
               msg_v, msg_v1, idx_all, sum_tbl, semm0, semm1):
    c = lax.axis_index("c")
    s = lax.axis_index("s")
    wid = s * _NC + c
    ew = e_tot // _NW
    rpt = npad // _NS
    kper = rpt // _CHUNK

    def zrow(r, carry):
        for j in range(d // 16):
            msg_v[r, pl.ds(j * 16, 16)] = jnp.zeros((16,), jnp.float32)
        return carry

    lax.fori_loop(0, _CHUNK, zrow, 0)
    for k in range(kper):
        sl = pl.ds(s * rpt + k * _CHUNK, _CHUNK)
        pltpu.sync_copy(msg_v, sum_tbl.at[sl])
    plsc.subcore_barrier()

    pltpu.sync_copy(tgt3_hbm.at[wid], idx_all)
    nchunk = ew // _CHUNK
    slots = ((msg_v, semm0), (msg_v1, semm1))

    def issue(k, slot):
        mv, sem = slot
        base = wid * ew + k * _CHUNK
        pltpu.async_copy(msg_hbm.at[pl.ds(base, _CHUNK)], mv, sem)

    def consume(k, slot):
        mv, sem = slot
        base = wid * ew + k * _CHUNK
        pltpu.make_async_copy(msg_hbm.at[pl.ds(base, _CHUNK)], mv, sem).wait()
        pltpu.sync_copy(mv, sum_tbl.at[idx_all.at[k]], add=True)

    issue(0, slots[0])
    issue(1, slots[1])

    def pair(g, carry):
        for b in range(2):
            k = g * 2 + b
            consume(k, slots[b])
            nk = k + 2

            @pl.when(nk < nchunk)
            def _():
                issue(nk, slots[b])
        return carry

    lax.fori_loop(0, nchunk // 2, pair, 0)
    if nchunk % 2 == 1:
        consume(nchunk - 1, slots[0])
    plsc.subcore_barrier()

    for k in range(kper):
        sl = pl.ds(s * rpt + k * _CHUNK, _CHUNK)
        osl = pl.ds(c * npad + s * rpt + k * _CHUNK, _CHUNK)
        pltpu.sync_copy(sum_tbl.at[sl], msg_v)
        pltpu.sync_copy(msg_v, sums_hbm.at[osl])


def _scatter_sums(msg, tgt3, npad):
    e_tot, d = msg.shape
    nchunk = tgt3.shape[1]
    mesh = plsc.VectorSubcoreMesh(core_axis_name="c", subcore_axis_name="s")
    kern = pl.kernel(
        functools.partial(_sums_body, e_tot, npad, d),
        out_type=jax.ShapeDtypeStruct((_NC * npad, d), jnp.float32),
        mesh=mesh,
        scratch_types=[
            pltpu.VMEM((_CHUNK, d), jnp.float32),
            pltpu.VMEM((_CHUNK, d), jnp.float32),
            pltpu.VMEM((nchunk, _CHUNK), jnp.int32),
            pltpu.VMEM_SHARED((npad, d), jnp.float32),
            pltpu.SemaphoreType.DMA,
            pltpu.SemaphoreType.DMA,
        ],
    )
    return kern(msg, tgt3)


def _cnts_body(e_tot, npad, d, tgt3_hbm, cnts_hbm, ones_v, idx_all, cnt_tbl,
               sem):
    c = lax.axis_index("c")
    s = lax.axis_index("s")
    wid = s * _NC + c
    ew = e_tot // _NW
    rpt = npad // _NS
    kper = rpt // _CHUNK
    one16 = jnp.full((16,), 1.0, jnp.float32)
    zero16 = jnp.zeros((16,), jnp.float32)

    def zrow(r, carry):
        for j in range(d // 16):
            ones_v[r, pl.ds(j * 16, 16)] = zero16
        return carry

    lax.fori_loop(0, _CHUNK, zrow, 0)
    for k in range(kper):
        sl = pl.ds(s * rpt + k * _CHUNK, _CHUNK)
        pltpu.sync_copy(ones_v, cnt_tbl.at[sl])

    def onerow(r, carry):
        for j in range(d // 16):
            ones_v[r, pl.ds(j * 16, 16)] = one16
        return carry

    lax.fori_loop(0, _CHUNK, onerow, 0)
    plsc.subcore_barrier()

    pltpu.sync_copy(tgt3_hbm.at[wid], idx_all)
    nchunk = ew // _CHUNK
    grp = 5

    def group_fn(g, carry):
        for b in range(grp):
            k = g * grp + b
            pltpu.async_copy(ones_v, cnt_tbl.at[idx_all.at[k]], sem,
                             add=True)
        for b in range(grp):
            k = g * grp + b
            pltpu.make_async_copy(ones_v, cnt_tbl.at[idx_all.at[k]],
                                  sem).wait()
        return carry

    lax.fori_loop(0, nchunk // grp, group_fn, 0)
    plsc.subcore_barrier()

    for k in range(kper):
        sl = pl.ds(s * rpt + k * _CHUNK, _CHUNK)
        osl = pl.ds(c * npad + s * rpt + k * _CHUNK, _CHUNK)
        pltpu.sync_copy(cnt_tbl.at[sl], ones_v)
        pltpu.sync_copy(ones_v, cnts_hbm.at[osl])


def _scatter_counts(tgt3, npad, d):
    e_tot = tgt3.shape[0] * tgt3.shape[1] * tgt3.shape[2]
    nchunk = tgt3.shape[1]
    mesh = plsc.VectorSubcoreMesh(core_axis_name="c", subcore_axis_name="s")
    kern = pl.kernel(
        functools.partial(_cnts_body, e_tot, npad, d),
        out_type=jax.ShapeDtypeStruct((_NC * npad, d), jnp.float32),
        mesh=mesh,
        scratch_types=[
            pltpu.VMEM((_CHUNK, d), jnp.float32),
            pltpu.VMEM((nchunk, _CHUNK), jnp.int32),
            pltpu.VMEM_SHARED((npad, d), jnp.float32),
            pltpu.SemaphoreType.DMA,
        ],
    )
    return kern(tgt3)


def _final_body(x_ref, s0_ref, s1_ref, c0_ref, c1_ref, o_ref):
    ssum = s0_ref[...] + s1_ref[...]
    cnt = c0_ref[...] + c1_ref[...]
    o_ref[...] = x_ref[...] + ssum / jnp.maximum(cnt, 1.0)


def _finalize(x, sums, cnts, npad):
    n, d = x.shape
    bn = 2000
    grid = ((n + bn - 1) // bn,)
    s0, s1 = sums[:npad], sums[npad:]
    c0, c1 = cnts[:npad], cnts[npad:]
    return pl.pallas_call(
        _final_body,
        grid=grid,
        in_specs=[
            pl.BlockSpec((bn, d), lambda i: (i, 0)),
            pl.BlockSpec((bn, d), lambda i: (i, 0)),
            pl.BlockSpec((bn, d), lambda i: (i, 0)),
            pl.BlockSpec((bn, d), lambda i: (i, 0)),
            pl.BlockSpec((bn, d), lambda i: (i, 0)),
        ],
        out_specs=pl.BlockSpec((bn, d), lambda i: (i, 0)),
        out_shape=jax.ShapeDtypeStruct((n, d), jnp.float32),
    )(x, s0, s1, c0, c1)


def kernel(x, edge_index, edge_feat, W1, b1, W2, b2, W3, b3):
    n, d = x.shape
    src = edge_index[0]
    dst = edge_index[1]
    w_ab = W1[: 2 * d, :]
    w1c = W1[2 * d:, :]
    npad = _npad_for(n)

    e_tot = src.shape[0]
    ew = e_tot // _NW
    src3 = src.reshape(_NW, ew // _CHUNK, _CHUNK)
    dst3 = dst.reshape(_NW, ew // _CHUNK, _CHUNK)

    xs, xt = _node_proj(x, w_ab)
    pre = _gather_add(xs, xt, src3, dst3)
    cnts = _scatter_counts(dst3, npad, d)
    msg = _edge_mlp(pre, edge_feat, w1c, b1, W2, b2, W3, b3)
    sums = _scatter_sums(msg, dst3, npad)
    return _finalize(x, sums, cnts, npad)
